# baseline (speedup 1.0000x reference)
import jax
import jax.numpy as jnp
from jax import lax
from jax.experimental import pallas as pl
from jax.experimental.pallas import tpu as pltpu
from jax.experimental.pallas import tpu_sc as plsc

_TOKENS = 16384
_HIDDEN = 4096
_EXPERTS = 64
_TOPK = 8
_BIG = 14336
_BT = 1024

_NC = 2
_NS = 16
_L = 16
_NW = _NC * _NS


def _mm_big_body(x_ref, w_ref, o_ref, c_ref):
    r = jnp.dot(x_ref[...], w_ref[...], preferred_element_type=jnp.float32)
    o_ref[...] = r
    c_ref[...] = r


def _mm_tail_body(x_ref, w_ref, li_ref, o_ref):
    del li_ref
    o_ref[...] = jnp.dot(x_ref[...], w_ref[...],
                         preferred_element_type=jnp.float32)


def _mm_big(hs, w_t):
    return pl.pallas_call(
        _mm_big_body,
        grid=(_BIG // _BT,),
        in_specs=[
            pl.BlockSpec((_BT, _HIDDEN), lambda i: (i, 0)),
            pl.BlockSpec((_HIDDEN, _EXPERTS), lambda i: (0, 0)),
        ],
        out_specs=[
            pl.BlockSpec((_BT, _EXPERTS), lambda i: (i, 0)),
            pl.BlockSpec((_BT, _EXPERTS), lambda i: (i, 0)),
        ],
        out_shape=[
            jax.ShapeDtypeStruct((_TOKENS, _EXPERTS), jnp.float32),
            jax.ShapeDtypeStruct((_BIG, _EXPERTS), jnp.float32),
        ],
    )(hs, w_t)


def _mm_tail(hs, w_t, logits_partial):
    tail = _TOKENS - _BIG
    return pl.pallas_call(
        _mm_tail_body,
        grid=(tail // _BT,),
        in_specs=[
            pl.BlockSpec((_BT, _HIDDEN), lambda i: (i + _BIG // _BT, 0)),
            pl.BlockSpec((_HIDDEN, _EXPERTS), lambda i: (0, 0)),
            pl.BlockSpec((_BT, _EXPERTS), lambda i: (i + _BIG // _BT, 0)),
        ],
        out_specs=pl.BlockSpec((_BT, _EXPERTS), lambda i: (i + _BIG // _BT, 0)),
        out_shape=jax.ShapeDtypeStruct((_TOKENS, _EXPERTS), jnp.float32),
        input_output_aliases={2: 0},
    )(hs, w_t, logits_partial)


def _make_topk_body(tpw, t0):
    def _topk_body(logits_hbm, scores_hbm, idx_hbm, slab, sc_v, ix_v):
        wid = lax.axis_index("s") * _NC + lax.axis_index("c")
        base = wid * tpw
        pltpu.sync_copy(logits_hbm.at[pl.ds(t0 + base, tpw)], slab)

        lane = lax.iota(jnp.int32, _L)
        topmask = lane < _TOPK

        @plsc.parallel_loop(0, tpw, 1, unroll=4)
        def body(i):
            v0 = slab[i, pl.ds(0, _L)]
            v1 = slab[i, pl.ds(_L, _L)]
            v2 = slab[i, pl.ds(2 * _L, _L)]
            v3 = slab[i, pl.ds(3 * _L, _L)]
            s0k, s0i = plsc.sort_key_val(v0, lane, descending=True)
            s1k, s1i = plsc.sort_key_val(v1, lane + _L, descending=False)
            s2k, s2i = plsc.sort_key_val(v2, lane + 2 * _L, descending=True)
            s3k, s3i = plsc.sort_key_val(v3, lane + 3 * _L, descending=False)
            m = s0k >= s1k
            l01k, l01i = plsc.sort_key_val(jnp.where(m, s0k, s1k),
                                           jnp.where(m, s0i, s1i),
                                           descending=True)
            m = s2k >= s3k
            l23k, l23i = plsc.sort_key_val(jnp.where(m, s2k, s3k),
                                           jnp.where(m, s2i, s3i),
                                           descending=False)
            m = l01k >= l23k
            fk, fi = plsc.sort_key_val(jnp.where(m, l01k, l23k),
                                       jnp.where(m, l01i, l23i),
                                       descending=True)
            mx = jnp.max(fk)
            e = jnp.where(topmask, jnp.exp(fk - mx), 0.0)
            s = jnp.sum(e)
            plsc.store_compressed(sc_v.at[pl.ds(i * _TOPK, _L)], e / s,
                                  mask=topmask)
            plsc.store_compressed(ix_v.at[pl.ds(i * _TOPK, _L)], fi,
                                  mask=topmask)

        flat = tpw * _TOPK
        pltpu.sync_copy(sc_v.at[pl.ds(0, flat)],
                        scores_hbm.at[pl.ds(base * _TOPK, flat)])
        pltpu.sync_copy(ix_v.at[pl.ds(0, flat)],
                        idx_hbm.at[pl.ds(base * _TOPK, flat)])

    return _topk_body


def _router_topk(logits, ntok, t0):
    tpw = ntok // _NW
    mesh = plsc.VectorSubcoreMesh(core_axis_name="c", subcore_axis_name="s",
                                  num_cores=_NC, num_subcores=_NS)
    fn = pl.kernel(
        _make_topk_body(tpw, t0),
        out_type=(
            jax.ShapeDtypeStruct((ntok * _TOPK,), jnp.float32),
            jax.ShapeDtypeStruct((ntok * _TOPK,), jnp.int32),
        ),
        mesh=mesh,
        compiler_params=pltpu.CompilerParams(needs_layout_passes=False),
        scratch_types=[
            pltpu.VMEM((tpw, _EXPERTS), jnp.float32),
            pltpu.VMEM((tpw * _TOPK + _L,), jnp.float32),
            pltpu.VMEM((tpw * _TOPK + _L,), jnp.int32),
        ],
    )
    return fn(logits)


def kernel(hidden_states, weight):
    hs = hidden_states.reshape(-1, _HIDDEN)
    w_t = weight.T
    logits_partial, lc_big = _mm_big(hs, w_t)
    sc_big, ix_big = _router_topk(lc_big, _BIG, 0)
    logits = _mm_tail(hs, w_t, logits_partial)
    sc_tail, ix_tail = _router_topk(logits, _TOKENS - _BIG, _BIG)
    scores = jnp.concatenate([sc_big, sc_tail], axis=0)
    idx = jnp.concatenate([ix_big, ix_tail], axis=0)
    return (logits,
            scores.reshape(_TOKENS, _TOPK),
            idx.reshape(_TOKENS, _TOPK))


# speedup vs baseline: 1.0961x; 1.0961x over previous
import jax
import jax.numpy as jnp
from jax import lax
from jax.experimental import pallas as pl
from jax.experimental.pallas import tpu as pltpu
from jax.experimental.pallas import tpu_sc as plsc

_TOKENS = 16384
_HIDDEN = 4096
_EXPERTS = 64
_TOPK = 8
_BT = 1024

_NC = 2
_NS = 16
_L = 16
_NW = _NC * _NS
_TPW = _TOKENS // _NW


def _logits_body(x_ref, w_ref, o_ref):
    o_ref[...] = jnp.dot(x_ref[...], w_ref[...],
                         preferred_element_type=jnp.float32)


def _compute_logits(hs, w_t):
    return pl.pallas_call(
        _logits_body,
        grid=(_TOKENS // _BT,),
        in_specs=[
            pl.BlockSpec((_BT, _HIDDEN), lambda i: (i, 0)),
            pl.BlockSpec((_HIDDEN, _EXPERTS), lambda i: (0, 0)),
        ],
        out_specs=pl.BlockSpec((_BT, _EXPERTS), lambda i: (i, 0)),
        out_shape=jax.ShapeDtypeStruct((_TOKENS, _EXPERTS), jnp.float32),
    )(hs, w_t)


def _topk_body(logits_hbm, scores_hbm, idx_hbm, slab, sc_v, ix_v):
    wid = lax.axis_index("s") * _NC + lax.axis_index("c")
    base = wid * _TPW
    pltpu.sync_copy(logits_hbm.at[pl.ds(base, _TPW)], slab)

    lane = lax.iota(jnp.int32, _L)
    topmask = lane < _TOPK

    @plsc.parallel_loop(0, _TPW, 1, unroll=4)
    def body(i):
        v0 = slab[i, pl.ds(0, _L)]
        v1 = slab[i, pl.ds(_L, _L)]
        v2 = slab[i, pl.ds(2 * _L, _L)]
        v3 = slab[i, pl.ds(3 * _L, _L)]
        s0k, s0i = plsc.sort_key_val(v0, lane, descending=True)
        s1k, s1i = plsc.sort_key_val(v1, lane + _L, descending=False)
        s2k, s2i = plsc.sort_key_val(v2, lane + 2 * _L, descending=True)
        s3k, s3i = plsc.sort_key_val(v3, lane + 3 * _L, descending=False)
        m = s0k >= s1k
        l01k, l01i = plsc.sort_key_val(jnp.where(m, s0k, s1k),
                                       jnp.where(m, s0i, s1i),
                                       descending=True)
        m = s2k >= s3k
        l23k, l23i = plsc.sort_key_val(jnp.where(m, s2k, s3k),
                                       jnp.where(m, s2i, s3i),
                                       descending=False)
        m = l01k >= l23k
        fk, fi = plsc.sort_key_val(jnp.where(m, l01k, l23k),
                                   jnp.where(m, l01i, l23i),
                                   descending=True)
        mx = jnp.max(fk)
        e = jnp.where(topmask, jnp.exp(fk - mx), 0.0)
        s = jnp.sum(e)
        plsc.store_compressed(sc_v.at[pl.ds(i * _TOPK, _L)], e / s,
                              mask=topmask)
        plsc.store_compressed(ix_v.at[pl.ds(i * _TOPK, _L)], fi,
                              mask=topmask)

    flat = _TPW * _TOPK
    pltpu.sync_copy(sc_v.at[pl.ds(0, flat)],
                    scores_hbm.at[pl.ds(base * _TOPK, flat)])
    pltpu.sync_copy(ix_v.at[pl.ds(0, flat)],
                    idx_hbm.at[pl.ds(base * _TOPK, flat)])


def _router_topk(logits):
    mesh = plsc.VectorSubcoreMesh(core_axis_name="c", subcore_axis_name="s",
                                  num_cores=_NC, num_subcores=_NS)
    fn = pl.kernel(
        _topk_body,
        out_type=(
            jax.ShapeDtypeStruct((_TOKENS * _TOPK,), jnp.float32),
            jax.ShapeDtypeStruct((_TOKENS * _TOPK,), jnp.int32),
        ),
        mesh=mesh,
        compiler_params=pltpu.CompilerParams(needs_layout_passes=False),
        scratch_types=[
            pltpu.VMEM((_TPW, _EXPERTS), jnp.float32),
            pltpu.VMEM((_TPW * _TOPK + _L,), jnp.float32),
            pltpu.VMEM((_TPW * _TOPK + _L,), jnp.int32),
        ],
    )
    return fn(logits)


def kernel(hidden_states, weight):
    hs = hidden_states.reshape(-1, _HIDDEN)
    logits = _compute_logits(hs, weight.T)
    scores_flat, idx_flat = _router_topk(logits)
    return (logits,
            scores_flat.reshape(_TOKENS, _TOPK),
            idx_flat.reshape(_TOKENS, _TOPK))


# speedup vs baseline: 1.1048x; 1.0080x over previous
import jax
import jax.numpy as jnp
from jax import lax
from jax.experimental import pallas as pl
from jax.experimental.pallas import tpu as pltpu
from jax.experimental.pallas import tpu_sc as plsc

_TOKENS = 16384
_HIDDEN = 4096
_EXPERTS = 64
_TOPK = 8
_BT = 512

_NC = 2
_NS = 16
_L = 16
_NW = _NC * _NS
_TPW = _TOKENS // _NW


def _logits_body(x_ref, w_ref, o_ref):
    o_ref[...] = jnp.dot(x_ref[...], w_ref[...],
                         preferred_element_type=jnp.float32)


def _compute_logits(hs, w_t):
    return pl.pallas_call(
        _logits_body,
        grid=(_TOKENS // _BT,),
        in_specs=[
            pl.BlockSpec((_BT, _HIDDEN), lambda i: (i, 0)),
            pl.BlockSpec((_HIDDEN, _EXPERTS), lambda i: (0, 0)),
        ],
        out_specs=pl.BlockSpec((_BT, _EXPERTS), lambda i: (i, 0)),
        out_shape=jax.ShapeDtypeStruct((_TOKENS, _EXPERTS), jnp.float32),
    )(hs, w_t)


def _topk_body(logits_hbm, scores_hbm, idx_hbm, slab, sc_v, ix_v):
    wid = lax.axis_index("s") * _NC + lax.axis_index("c")
    base = wid * _TPW
    pltpu.sync_copy(logits_hbm.at[pl.ds(base, _TPW)], slab)

    lane = lax.iota(jnp.int32, _L)
    topmask = lane < _TOPK

    @plsc.parallel_loop(0, _TPW, 1, unroll=4)
    def body(i):
        v0 = slab[i, pl.ds(0, _L)]
        v1 = slab[i, pl.ds(_L, _L)]
        v2 = slab[i, pl.ds(2 * _L, _L)]
        v3 = slab[i, pl.ds(3 * _L, _L)]
        s0k, s0i = plsc.sort_key_val(v0, lane, descending=True)
        s1k, s1i = plsc.sort_key_val(v1, lane + _L, descending=False)
        s2k, s2i = plsc.sort_key_val(v2, lane + 2 * _L, descending=True)
        s3k, s3i = plsc.sort_key_val(v3, lane + 3 * _L, descending=False)
        m = s0k >= s1k
        l01k, l01i = plsc.sort_key_val(jnp.where(m, s0k, s1k),
                                       jnp.where(m, s0i, s1i),
                                       descending=True)
        m = s2k >= s3k
        l23k, l23i = plsc.sort_key_val(jnp.where(m, s2k, s3k),
                                       jnp.where(m, s2i, s3i),
                                       descending=False)
        m = l01k >= l23k
        fk, fi = plsc.sort_key_val(jnp.where(m, l01k, l23k),
                                   jnp.where(m, l01i, l23i),
                                   descending=True)
        mx = jnp.max(fk)
        e = jnp.where(topmask, jnp.exp(fk - mx), 0.0)
        s = jnp.sum(e)
        plsc.store_compressed(sc_v.at[pl.ds(i * _TOPK, _L)], e / s,
                              mask=topmask)
        plsc.store_compressed(ix_v.at[pl.ds(i * _TOPK, _L)], fi,
                              mask=topmask)

    flat = _TPW * _TOPK
    pltpu.sync_copy(sc_v.at[pl.ds(0, flat)],
                    scores_hbm.at[pl.ds(base * _TOPK, flat)])
    pltpu.sync_copy(ix_v.at[pl.ds(0, flat)],
                    idx_hbm.at[pl.ds(base * _TOPK, flat)])


def _router_topk(logits):
    mesh = plsc.VectorSubcoreMesh(core_axis_name="c", subcore_axis_name="s",
                                  num_cores=_NC, num_subcores=_NS)
    fn = pl.kernel(
        _topk_body,
        out_type=(
            jax.ShapeDtypeStruct((_TOKENS * _TOPK,), jnp.float32),
            jax.ShapeDtypeStruct((_TOKENS * _TOPK,), jnp.int32),
        ),
        mesh=mesh,
        compiler_params=pltpu.CompilerParams(needs_layout_passes=False),
        scratch_types=[
            pltpu.VMEM((_TPW, _EXPERTS), jnp.float32),
            pltpu.VMEM((_TPW * _TOPK + _L,), jnp.float32),
            pltpu.VMEM((_TPW * _TOPK + _L,), jnp.int32),
        ],
    )
    return fn(logits)


def kernel(hidden_states, weight):
    hs = hidden_states.reshape(-1, _HIDDEN)
    logits = _compute_logits(hs, weight.T)
    scores_flat, idx_flat = _router_topk(logits)
    return (logits,
            scores_flat.reshape(_TOKENS, _TOPK),
            idx_flat.reshape(_TOKENS, _TOPK))
